# baseline (speedup 1.0000x reference)
import jax
import jax.numpy as jnp
from jax import lax
from jax.experimental import pallas as pl
from jax.experimental.pallas import tpu as pltpu
from jax.experimental.pallas import tpu_sc as plsc

V = 100000
D = 128
B = 1024
L = 200
N = B * L

NCHK = 4
BH = B // NCHK
NH = BH * L

NC = 2
NS = 16
NW = NC * NS
ROWS_PER_W = NH // NW
CHUNK = 160
NCHUNK = ROWS_PER_W // CHUNK
UNROLL = 4


def _gather_body(cw_hbm, cn_hbm, ctab_hbm, xtab_hbm, vc_hbm, uc_hbm,
                 idx_v, buf_a, buf_b, bf_a, bf_b, sem_a, sem_b, wsem_a, wsem_b):
    wid = lax.axis_index("s") * NC + lax.axis_index("c")
    base = wid * ROWS_PER_W
    bufs = (buf_a, buf_b)
    sems = (sem_a, sem_b)
    bfs = (bf_a, bf_b)
    wsems = (wsem_a, wsem_b)

    def one_table(idx_hbm, tab_hbm, out_hbm):
        pltpu.sync_copy(idx_hbm.at[pl.ds(base, ROWS_PER_W)], idx_v)

        def start(j):
            idx_slice = idx_v.at[pl.ds(j * CHUNK, CHUNK)]
            return pltpu.async_copy(tab_hbm.at[idx_slice], bufs[j % 2],
                                    sems[j % 2])

        def convert(buf, bfb):
            @plsc.parallel_loop(0, CHUNK, step=1, unroll=UNROLL)
            def _row(r):
                for g in range(4):
                    a = buf[r, pl.ds(g * 32, 16)]
                    b = buf[r, pl.ds(g * 32 + 16, 16)]
                    p = plsc.pack(a, b, format=plsc.PackFormat.INTERLEAVED)
                    bfb[r, pl.ds(g * 16, 16)] = plsc.bitcast(p, jnp.int32)

        pending = start(0)
        for j in range(NCHUNK):
            pending.wait()
            if j + 1 < NCHUNK:
                pending = start(j + 1)
            convert(bufs[j % 2], bfs[j % 2])
            pltpu.sync_copy(bfs[j % 2],
                            out_hbm.at[pl.ds(base + j * CHUNK, CHUNK)])

    one_table(cw_hbm, ctab_hbm, vc_hbm)
    one_table(cn_hbm, xtab_hbm, uc_hbm)


_gather = pl.kernel(
    _gather_body,
    out_type=(
        jax.ShapeDtypeStruct((NH, D // 2), jnp.int32),
        jax.ShapeDtypeStruct((NH, D // 2), jnp.int32),
    ),
    mesh=plsc.VectorSubcoreMesh(core_axis_name="c", subcore_axis_name="s"),
    compiler_params=pltpu.CompilerParams(needs_layout_passes=False),
    scratch_types=[
        pltpu.VMEM((ROWS_PER_W,), jnp.int32),
        pltpu.VMEM((CHUNK, D), jnp.float32),
        pltpu.VMEM((CHUNK, D), jnp.float32),
        pltpu.VMEM((CHUNK, D // 2), jnp.int32),
        pltpu.VMEM((CHUNK, D // 2), jnp.int32),
        pltpu.SemaphoreType.DMA,
        pltpu.SemaphoreType.DMA,
        pltpu.SemaphoreType.DMA,
        pltpu.SemaphoreType.DMA,
    ],
)

BG = 32
HSTEPS = BH // BG


def _bmm_first_body(v_ref, u_ref, o_ref):
    for i in range(BG):
        o_ref[i] = lax.dot_general(v_ref[i], u_ref[i],
                                   (((1,), (1,)), ((), ())),
                                   preferred_element_type=jnp.float32)


def _bmm_chunk_body(full_ref, v_ref, u_ref, o_ref):
    _bmm_first_body(v_ref, u_ref, o_ref)


def _bmm_first(vc, uc):
    return pl.pallas_call(
        _bmm_first_body,
        grid=(HSTEPS,),
        in_specs=[
            pl.BlockSpec((BG, L, D), lambda b: (b, 0, 0)),
            pl.BlockSpec((BG, L, D), lambda b: (b, 0, 0)),
        ],
        out_specs=pl.BlockSpec((BG, L, L), lambda b: (b, 0, 0)),
        out_shape=jax.ShapeDtypeStruct((B, L, L), jnp.float32),
    )(vc, uc)


def _bmm_chunk(full, vc, uc, c):
    return pl.pallas_call(
        _bmm_chunk_body,
        grid=(HSTEPS,),
        in_specs=[
            pl.BlockSpec(memory_space=pl.ANY),
            pl.BlockSpec((BG, L, D), lambda b: (b, 0, 0)),
            pl.BlockSpec((BG, L, D), lambda b: (b, 0, 0)),
        ],
        out_specs=pl.BlockSpec((BG, L, L), lambda b, c=c: (b + c * HSTEPS, 0, 0)),
        out_shape=jax.ShapeDtypeStruct((B, L, L), jnp.float32),
        input_output_aliases={0: 0},
    )(full, vc, uc)


def _unpack(words):
    bf = lax.bitcast_convert_type(words, jnp.bfloat16)
    return bf.reshape(BH, L, D)


def kernel(center_words, context_negatives, center_table, context_table):
    cw = center_words.reshape(NCHK, NH)
    cn = context_negatives.reshape(NCHK, NH)
    gathered = [_gather(cw[c], cn[c], center_table, context_table)
                for c in range(NCHK)]
    full = _bmm_first(_unpack(gathered[0][0]), _unpack(gathered[0][1]))
    for c in range(1, NCHK):
        full = _bmm_chunk(full, _unpack(gathered[c][0]),
                          _unpack(gathered[c][1]), c)
    return full


# speedup vs baseline: 3.1733x; 3.1733x over previous
import jax
import jax.numpy as jnp
from jax import lax
from jax.experimental import pallas as pl
from jax.experimental.pallas import tpu as pltpu
from jax.experimental.pallas import tpu_sc as plsc

V = 100000
D = 128
B = 1024
L = 200
N = B * L

NCHK = 8
BH = B // NCHK
NH = BH * L

NC = 2
NS = 16
NW = NC * NS
ROWS_PER_W = NH // NW
CHUNK = 400
NCHUNK = ROWS_PER_W // CHUNK


def _gather_body(cw_hbm, cn_hbm, ctab_hbm, xtab_hbm, vc_hbm, uc_hbm,
                 idx_v, buf_a, buf_b, sem_a, sem_b):
    wid = lax.axis_index("s") * NC + lax.axis_index("c")
    base = wid * ROWS_PER_W
    bufs = (buf_a, buf_b)
    sems = (sem_a, sem_b)

    def one_table(idx_hbm, tab_hbm, out_hbm):
        pltpu.sync_copy(idx_hbm.at[pl.ds(base, ROWS_PER_W)], idx_v)

        def start(j):
            idx_slice = idx_v.at[pl.ds(j * CHUNK, CHUNK)]
            return pltpu.async_copy(tab_hbm.at[idx_slice], bufs[j % 2],
                                    sems[j % 2])

        pending = start(0)
        for j in range(NCHUNK):
            pending.wait()
            if j + 1 < NCHUNK:
                pending = start(j + 1)
            pltpu.sync_copy(bufs[j % 2],
                            out_hbm.at[pl.ds(base + j * CHUNK, CHUNK)])

    one_table(cw_hbm, ctab_hbm, vc_hbm)
    one_table(cn_hbm, xtab_hbm, uc_hbm)


_gather = pl.kernel(
    _gather_body,
    out_type=(
        jax.ShapeDtypeStruct((NH, D), jnp.float32),
        jax.ShapeDtypeStruct((NH, D), jnp.float32),
    ),
    mesh=plsc.VectorSubcoreMesh(core_axis_name="c", subcore_axis_name="s"),
    scratch_types=[
        pltpu.VMEM((ROWS_PER_W,), jnp.int32),
        pltpu.VMEM((CHUNK, D), jnp.float32),
        pltpu.VMEM((CHUNK, D), jnp.float32),
        pltpu.SemaphoreType.DMA,
        pltpu.SemaphoreType.DMA,
    ],
)

BG = 32
HSTEPS = BH // BG


def _bmm_first_body(v_ref, u_ref, o_ref):
    for i in range(BG):
        v = v_ref[i].astype(jnp.bfloat16)
        u = u_ref[i].astype(jnp.bfloat16)
        o_ref[i] = lax.dot_general(v, u, (((1,), (1,)), ((), ())),
                                   preferred_element_type=jnp.float32)


def _bmm_chunk_body(full_ref, v_ref, u_ref, o_ref):
    _bmm_first_body(v_ref, u_ref, o_ref)


def _bmm_first(vc, uc):
    return pl.pallas_call(
        _bmm_first_body,
        grid=(HSTEPS,),
        in_specs=[
            pl.BlockSpec((BG, L, D), lambda b: (b, 0, 0)),
            pl.BlockSpec((BG, L, D), lambda b: (b, 0, 0)),
        ],
        out_specs=pl.BlockSpec((BG, L, L), lambda b: (b, 0, 0)),
        out_shape=jax.ShapeDtypeStruct((B, L, L), jnp.float32),
    )(vc, uc)


def _bmm_chunk(full, vc, uc, c):
    return pl.pallas_call(
        _bmm_chunk_body,
        grid=(HSTEPS,),
        in_specs=[
            pl.BlockSpec(memory_space=pl.ANY),
            pl.BlockSpec((BG, L, D), lambda b: (b, 0, 0)),
            pl.BlockSpec((BG, L, D), lambda b: (b, 0, 0)),
        ],
        out_specs=pl.BlockSpec((BG, L, L), lambda b, c=c: (b + c * HSTEPS, 0, 0)),
        out_shape=jax.ShapeDtypeStruct((B, L, L), jnp.float32),
        input_output_aliases={0: 0},
    )(full, vc, uc)


def kernel(center_words, context_negatives, center_table, context_table):
    cw = center_words.reshape(NCHK, NH)
    cn = context_negatives.reshape(NCHK, NH)
    gathered = [_gather(cw[c], cn[c], center_table, context_table)
                for c in range(NCHK)]
    full = _bmm_first(gathered[0][0].reshape(BH, L, D),
                      gathered[0][1].reshape(BH, L, D))
    for c in range(1, NCHK):
        full = _bmm_chunk(full,
                          gathered[c][0].reshape(BH, L, D),
                          gathered[c][1].reshape(BH, L, D), c)
    return full


# speedup vs baseline: 3.2653x; 1.0290x over previous
import jax
import jax.numpy as jnp
from jax import lax
from jax.experimental import pallas as pl
from jax.experimental.pallas import tpu as pltpu
from jax.experimental.pallas import tpu_sc as plsc

V = 100000
D = 128
B = 1024
L = 200
N = B * L

NCHK = 8
BH = B // NCHK
NH = BH * L

NC = 2
NS = 16
NW = NC * NS
ROWS_PER_W = NH // NW
CHUNK = 400
NCHUNK = ROWS_PER_W // CHUNK


def _gather_body(cw_hbm, cn_hbm, ctab_hbm, xtab_hbm, vc_hbm, uc_hbm,
                 idx_v, buf_a, buf_b, sem_a, sem_b):
    wid = lax.axis_index("s") * NC + lax.axis_index("c")
    base = wid * ROWS_PER_W
    bufs = (buf_a, buf_b)
    sems = (sem_a, sem_b)

    def one_table(idx_hbm, tab_hbm, out_hbm):
        pltpu.sync_copy(idx_hbm.at[pl.ds(base, ROWS_PER_W)], idx_v)

        def start(j):
            idx_slice = idx_v.at[pl.ds(j * CHUNK, CHUNK)]
            return pltpu.async_copy(tab_hbm.at[idx_slice], bufs[j % 2],
                                    sems[j % 2])

        pending = start(0)
        for j in range(NCHUNK):
            pending.wait()
            if j + 1 < NCHUNK:
                pending = start(j + 1)
            pltpu.sync_copy(bufs[j % 2],
                            out_hbm.at[pl.ds(base + j * CHUNK, CHUNK)])

    one_table(cw_hbm, ctab_hbm, vc_hbm)
    one_table(cn_hbm, xtab_hbm, uc_hbm)


_gather = pl.kernel(
    _gather_body,
    out_type=(
        jax.ShapeDtypeStruct((NH, D), jnp.float32),
        jax.ShapeDtypeStruct((NH, D), jnp.float32),
    ),
    mesh=plsc.VectorSubcoreMesh(core_axis_name="c", subcore_axis_name="s"),
    scratch_types=[
        pltpu.VMEM((ROWS_PER_W,), jnp.int32),
        pltpu.VMEM((CHUNK, D), jnp.float32),
        pltpu.VMEM((CHUNK, D), jnp.float32),
        pltpu.SemaphoreType.DMA,
        pltpu.SemaphoreType.DMA,
    ],
)

BG = 64
HSTEPS = BH // BG


def _bmm_first_body(v_ref, u_ref, o_ref):
    for i in range(BG):
        v = v_ref[i].astype(jnp.bfloat16)
        u = u_ref[i].astype(jnp.bfloat16)
        o_ref[i] = lax.dot_general(v, u, (((1,), (1,)), ((), ())),
                                   preferred_element_type=jnp.float32)


def _bmm_chunk_body(full_ref, v_ref, u_ref, o_ref):
    _bmm_first_body(v_ref, u_ref, o_ref)


def _bmm_first(vc, uc):
    return pl.pallas_call(
        _bmm_first_body,
        grid=(HSTEPS,),
        in_specs=[
            pl.BlockSpec((BG, L, D), lambda b: (b, 0, 0)),
            pl.BlockSpec((BG, L, D), lambda b: (b, 0, 0)),
        ],
        out_specs=pl.BlockSpec((BG, L, L), lambda b: (b, 0, 0)),
        out_shape=jax.ShapeDtypeStruct((B, L, L), jnp.float32),
    )(vc, uc)


def _bmm_chunk(full, vc, uc, c):
    return pl.pallas_call(
        _bmm_chunk_body,
        grid=(HSTEPS,),
        in_specs=[
            pl.BlockSpec(memory_space=pl.ANY),
            pl.BlockSpec((BG, L, D), lambda b: (b, 0, 0)),
            pl.BlockSpec((BG, L, D), lambda b: (b, 0, 0)),
        ],
        out_specs=pl.BlockSpec((BG, L, L), lambda b, c=c: (b + c * HSTEPS, 0, 0)),
        out_shape=jax.ShapeDtypeStruct((B, L, L), jnp.float32),
        input_output_aliases={0: 0},
    )(full, vc, uc)


def kernel(center_words, context_negatives, center_table, context_table):
    cw = center_words.reshape(NCHK, NH)
    cn = context_negatives.reshape(NCHK, NH)
    gathered = [_gather(cw[c], cn[c], center_table, context_table)
                for c in range(NCHK)]
    full = _bmm_first(gathered[0][0].reshape(BH, L, D),
                      gathered[0][1].reshape(BH, L, D))
    for c in range(1, NCHK):
        full = _bmm_chunk(full,
                          gathered[c][0].reshape(BH, L, D),
                          gathered[c][1].reshape(BH, L, D), c)
    return full
